# Initial kernel scaffold; baseline (speedup 1.0000x reference)
import jax
import jax.numpy as jnp
from jax.experimental import pallas as pl


def kernel(inputs, token_table, pos_table):
    raise NotImplementedError("write your pallas kernel here")


# speedup vs baseline: 1.4939x; 1.4939x over previous
import functools
import math

import jax
import jax.numpy as jnp
from jax import lax
from jax.experimental import pallas as pl
from jax.experimental.pallas import tpu as pltpu
from jax.experimental.pallas import tpu_sc as plsc

SEQ = 200
DIM = 32
HALF = 16
NC, NS = 2, 16
NW = NC * NS
SCALE = float(math.sqrt(float(DIM)))

CHUNK = 800
NBUF = 2
REPS = CHUNK // SEQ


def _body(idx_hbm, tok_hbm, pos_hbm, out_hbm,
          idx_v, pos_v, bufs, gsems, osems, *, rows_per_w, n_chunks):
    wid = lax.axis_index("s") * NC + lax.axis_index("c")
    base = wid * rows_per_w

    pltpu.sync_copy(idx_hbm.at[pl.ds(base, rows_per_w)], idx_v)
    pltpu.sync_copy(pos_hbm, pos_v)

    def start_gather(g, b):
        pltpu.async_copy(
            tok_hbm.at[idx_v.at[pl.ds(g * CHUNK, CHUNK)]], bufs[b], gsems[b])

    def wait_gather(g, b):
        pltpu.make_async_copy(
            tok_hbm.at[idx_v.at[pl.ds(g * CHUNK, CHUNK)]], bufs[b],
            gsems[b]).wait()

    def start_out(g, b):
        pltpu.async_copy(
            bufs[b], out_hbm.at[pl.ds(base + g * CHUNK, CHUNK)], osems[b])

    def wait_out(g, b):
        pltpu.make_async_copy(
            bufs[b], out_hbm.at[pl.ds(base + g * CHUNK, CHUNK)],
            osems[b]).wait()

    def compute(b):
        buf = bufs[b]

        def phase(p, _):
            plo = pos_v[p, pl.ds(0, HALF)]
            phi = pos_v[p, pl.ds(HALF, HALF)]
            for r in range(REPS):
                row = r * SEQ + p
                buf[row, pl.ds(0, HALF)] = buf[row, pl.ds(0, HALF)] * SCALE + plo
                buf[row, pl.ds(HALF, HALF)] = (
                    buf[row, pl.ds(HALF, HALF)] * SCALE + phi)
            return 0

        lax.fori_loop(0, SEQ, phase, 0)

    start_gather(0, 0)

    def iter_body(it, _):
        for b in range(NBUF):
            g = it * NBUF + b
            wait_gather(g, b)
            nb = (b + 1) % NBUF

            @pl.when(g + 1 < n_chunks)
            def _():
                @pl.when(g >= 1)
                def _():
                    wait_out(g - 1, nb)
                start_gather(g + 1, nb)

            compute(b)
            start_out(g, b)
        return 0

    lax.fori_loop(0, n_chunks // NBUF, iter_body, 0)
    wait_out(n_chunks - 2, (n_chunks - 2) % NBUF)
    wait_out(n_chunks - 1, (n_chunks - 1) % NBUF)


def kernel(inputs, token_table, pos_table):
    B, S = inputs.shape
    assert S == SEQ and token_table.shape[1] == DIM
    total = B * S
    rows_per_w = total // NW
    n_chunks = rows_per_w // CHUNK
    assert rows_per_w % CHUNK == 0 and n_chunks % NBUF == 0

    idx_flat = inputs.reshape(total).astype(jnp.int32)

    mesh = plsc.VectorSubcoreMesh(core_axis_name="c", subcore_axis_name="s")
    run = functools.partial(
        pl.kernel,
        out_type=jax.ShapeDtypeStruct((total, DIM), jnp.float32),
        mesh=mesh,
        compiler_params=pltpu.CompilerParams(use_tc_tiling_on_sc=False),
        scratch_types=[
            pltpu.VMEM((rows_per_w,), jnp.int32),
            pltpu.VMEM((SEQ, DIM), jnp.float32),
            [pltpu.VMEM((CHUNK, DIM), jnp.float32) for _ in range(NBUF)],
            [pltpu.SemaphoreType.DMA for _ in range(NBUF)],
            [pltpu.SemaphoreType.DMA for _ in range(NBUF)],
        ],
    )(functools.partial(_body, rows_per_w=rows_per_w, n_chunks=n_chunks))

    out = run(idx_flat, token_table, pos_table)
    return out.reshape(B, S, DIM)
